# baseline (speedup 1.0000x reference)
import functools

import jax
import jax.numpy as jnp
from jax import lax
from jax.experimental import pallas as pl
from jax.experimental.pallas import tpu as pltpu
from jax.experimental.pallas import tpu_sc as plsc

N = 10000
E = 320000
D = 128
B = 256
V = 100000
SCALE = 12.0

NC = 2
NS = 16
NW = NC * NS

N_PAD = 10240
SLAB = N_PAD // NS
GCHUNK = 80
EDGES_PER_TILE = E // NS
NCHUNK = EDGES_PER_TILE // GCHUNK

_MESH = dict(core_axis_name="c", subcore_axis_name="s", num_cores=NC,
             num_subcores=NS)
_SC_PARAMS = pltpu.CompilerParams(needs_layout_passes=False)
_SC_PARAMS_NT = pltpu.CompilerParams(needs_layout_passes=False,
                                     use_tc_tiling_on_sc=False)

_GDN = lax.GatherDimensionNumbers(offset_dims=(), collapsed_slice_dims=(0,),
                                  start_index_map=(0,))


def _bcast_lane(v16, j):
    idx = jnp.full((16, 1), j, jnp.int32)
    return lax.gather(v16, idx, _GDN, (1,),
                      mode=lax.GatherScatterMode.PROMISE_IN_BOUNDS)


def _feat_gather(iid, emb):
    per_w = N_PAD // NW
    nch = per_w // GCHUNK

    @functools.partial(
        pl.kernel,
        out_type=jax.ShapeDtypeStruct((N, D), jnp.float32),
        mesh=plsc.VectorSubcoreMesh(**_MESH),
        compiler_params=_SC_PARAMS,
        scratch_types=[
            pltpu.VMEM((GCHUNK,), jnp.int32),
            pltpu.VMEM((GCHUNK, D), jnp.float32),
            pltpu.SemaphoreType.DMA,
        ],
    )
    def k(iid_hbm, emb_hbm, feat_hbm, idx_v, rows_v, sem):
        wid = lax.axis_index("s") * NC + lax.axis_index("c")
        base_w = jnp.minimum(wid * per_w, N - per_w)

        def body(j, carry):
            base = base_w + j * GCHUNK
            pltpu.sync_copy(iid_hbm.at[pl.ds(base, GCHUNK)], idx_v)
            pltpu.async_copy(emb_hbm.at[idx_v], rows_v, sem).wait()
            pltpu.sync_copy(rows_v, feat_hbm.at[pl.ds(base, GCHUNK)])
            return carry

        lax.fori_loop(0, nch, body, 0)

    return k(iid, emb)


def _edge_agg(feat, eidx_flat, edge_weight):
    L = 16
    NH = N_PAD // 2
    CG = D // L
    K = 640
    SUP = 2 * K
    NCH = E // K
    GSUB = 128

    featg = feat.reshape(N, CG, L).transpose(1, 0, 2).reshape(CG * N, L)
    zacc = jnp.zeros((NH, L), jnp.float32)
    zden = jnp.zeros((N_PAD // D, D), jnp.float32)

    @functools.partial(
        pl.kernel,
        out_type=(jax.ShapeDtypeStruct((NC, NS, NH, L), jnp.float32),
                  jax.ShapeDtypeStruct((NC, N_PAD // D, D), jnp.float32)),
        mesh=plsc.VectorSubcoreMesh(**_MESH),
        compiler_params=_SC_PARAMS_NT,
        scratch_types=[
            pltpu.VMEM((SUP,), jnp.int32),
            pltpu.VMEM((SUP,), jnp.int32),
            pltpu.VMEM((SUP,), jnp.float32),
            pltpu.VMEM((SUP,), jnp.int32),
            pltpu.VMEM((SUP,), jnp.int32),
            pltpu.VMEM((SUP,), jnp.float32),
            pltpu.VMEM((K, L), jnp.float32),
            pltpu.VMEM((K, L), jnp.float32),
            pltpu.VMEM((NH, L), jnp.float32),
            pltpu.VMEM((N_PAD // D, D), jnp.float32),
            pltpu.SemaphoreType.DMA,
            pltpu.SemaphoreType.DMA,
        ],
    )
    def k(featg_hbm, eidx_hbm, ew_hbm, zacc_hbm, zden_hbm, num_hbm, den_hbm,
          src_a, dst_a, w_a, src_b, dst_b, w_b, rows_a, rows_b,
          acc_v, den_v, sem_a, sem_b):
        c = lax.axis_index("c")
        s = lax.axis_index("s")
        g = s // 2
        h = s % 2
        lo = h * NH
        gN = g * N
        iota16 = lax.iota(jnp.int32, 16)

        pltpu.sync_copy(zacc_hbm, acc_v)
        pltpu.sync_copy(zden_hbm, den_v)

        def superload(sup, src_v, dst_v, w_v):
            base = sup * SUP
            pltpu.sync_copy(eidx_hbm.at[pl.ds(c * E + base, SUP)], src_v)
            pltpu.sync_copy(eidx_hbm.at[pl.ds((1 - c) * E + base, SUP)], dst_v)
            pltpu.sync_copy(ew_hbm.at[pl.ds(base, SUP)], w_v)

            def off(j, carry):
                sl = pl.ds(j * L, L)
                src_v[sl] = src_v[sl] + gN
                return carry

            lax.fori_loop(0, SUP // L, off, 0)

        def issue(chunk, src_v, rows, sem):
            o = (chunk % 2) * K
            for j in range(K // GSUB):
                pltpu.async_copy(
                    featg_hbm.at[src_v.at[pl.ds(o + j * GSUB, GSUB)]],
                    rows.at[pl.ds(j * GSUB, GSUB)], sem)

        def drain(src_v, rows, sem):
            for j in range(K // GSUB):
                pltpu.make_async_copy(
                    featg_hbm.at[src_v.at[pl.ds(j * GSUB, GSUB)]],
                    rows.at[pl.ds(j * GSUB, GSUB)], sem).wait()

        def process(chunk, dst_v, w_v, rows):
            o = (chunk % 2) * K

            def grp(g2, carry):
                off = o + g2 * L
                d16 = dst_v[pl.ds(off, L)]
                w16 = w_v[pl.ds(off, L)]
                dstl = d16 - lo
                msk = (d16 >= lo) & (dstl < NH)
                erow = iota16 + g2 * L
                for col in range(L):
                    cold = jnp.full((16,), col, jnp.int32)
                    val = plsc.load_gather(rows, [erow, cold])
                    plsc.addupdate_scatter(acc_v, [dstl, cold], val * w16,
                                           mask=msk)

                @pl.when(s == 0)
                def _():
                    plsc.addupdate_scatter(
                        den_v, [d16 >> 7, d16 & (D - 1)], w16)

                return carry

            lax.fori_loop(0, K // L, grp, 0)

        SA = (src_a, dst_a, w_a)
        SB = (src_b, dst_b, w_b)

        def phase(chunk, cur_set, nxt_set, rows_c, sem_c, rows_n, sem_n,
                  load_next_super):
            nxt = chunk + 1

            @pl.when(nxt < NCH)
            def _():
                if load_next_super:
                    superload(nxt // 2, *nxt_set)
                issue(nxt, nxt_set[0], rows_n, sem_n)

            drain(cur_set[0], rows_c, sem_c)
            process(chunk, cur_set[1], cur_set[2], rows_c)

        superload(0, *SA)
        issue(0, src_a, rows_a, sem_a)

        def quad(i, carry):
            n0 = 4 * i
            phase(n0, SA, SA, rows_a, sem_a, rows_b, sem_b, False)
            phase(n0 + 1, SA, SB, rows_b, sem_b, rows_a, sem_a, True)
            phase(n0 + 2, SB, SB, rows_a, sem_a, rows_b, sem_b, False)
            phase(n0 + 3, SB, SA, rows_b, sem_b, rows_a, sem_a, True)
            return carry

        lax.fori_loop(0, NCH // 4, quad, 0)

        pltpu.sync_copy(acc_v, num_hbm.at[c, s])

        @pl.when(s == 0)
        def _():
            pltpu.sync_copy(den_v, den_hbm.at[c])

    num5, den2 = k(featg, eidx_flat, edge_weight, zacc, zden)
    num = (num5.reshape(NC, CG, 2, NH, L).transpose(0, 2, 3, 1, 4)
           .reshape(NC, N_PAD, D))
    return num, den2.reshape(NC, N_PAD)


def _gru(num1, den1, num2, den2, feat, W1, W2, w_ih, w_hh, b_ih, b_hh):
    nb = 2000
    cdim = (((1,), (1,)), ((), ()))

    def body(n1_ref, d1_ref, n2_ref, d2_ref, f_ref, W1_ref, W2_ref,
             wih_ref, whh_ref, bih_ref, bhh_ref, o_ref):
        d1 = d1_ref[...]
        d2 = d2_ref[...]
        h1 = jnp.where(d1 > 0, n1_ref[...] / jnp.maximum(d1, 1e-12), 0.0)
        h2 = jnp.where(d2 > 0, n2_ref[...] / jnp.maximum(d2, 1e-12), 0.0)
        neigh1 = lax.dot_general(h1, W1_ref[...], cdim,
                                 preferred_element_type=jnp.float32)
        neigh2 = lax.dot_general(h2, W2_ref[...], cdim,
                                 preferred_element_type=jnp.float32)
        wih = wih_ref[...]
        gi = (lax.dot_general(neigh1, wih[:, :D], cdim,
                              preferred_element_type=jnp.float32)
              + lax.dot_general(neigh2, wih[:, D:], cdim,
                                preferred_element_type=jnp.float32)
              + bih_ref[...])
        f = f_ref[...]
        gh = lax.dot_general(f, whh_ref[...], cdim,
                             preferred_element_type=jnp.float32) + bhh_ref[...]
        r = jax.nn.sigmoid(gi[:, :D] + gh[:, :D])
        z = jax.nn.sigmoid(gi[:, D:2 * D] + gh[:, D:2 * D])
        ng = jnp.tanh(gi[:, 2 * D:] + r * gh[:, 2 * D:])
        o_ref[...] = (1.0 - z) * ng + z * f

    full = lambda shape: pl.BlockSpec(shape, lambda i: (0, 0))
    blk = lambda shape: pl.BlockSpec(shape, lambda i: (i, 0))
    return pl.pallas_call(
        body,
        grid=(N // nb,),
        in_specs=[blk((nb, D)), blk((nb, 1)), blk((nb, D)), blk((nb, 1)),
                  blk((nb, D)), full((D, D)), full((D, D)),
                  full((3 * D, 2 * D)), full((3 * D, D)),
                  full((1, 3 * D)), full((1, 3 * D))],
        out_specs=blk((nb, D)),
        out_shape=jax.ShapeDtypeStruct((N, D), jnp.float32),
    )(num1, den1, num2, den2, feat, W1, W2, w_ih, w_hh, b_ih, b_hh)


def _readout(feat, seg2d, last2d, fc_u_W, fc_v_W, fc_v_b, fc_e_W, fc_sr_W):
    ct11 = (((1,), (1,)), ((), ()))
    ct00 = (((0,), (0,)), ((), ()))
    ct10 = (((1,), (0,)), ((), ()))

    def body(f_ref, seg_ref, last_ref, u_ref, v_ref, vb_ref, e_ref, sr_ref,
             o_ref):
        f = f_ref[...]
        S = (seg_ref[...] == lax.broadcasted_iota(jnp.int32, (N, B), 1)
             ).astype(jnp.float32)
        L = (last_ref[...] == lax.broadcasted_iota(jnp.int32, (B, N), 1)
             ).astype(jnp.float32)
        sr_l = lax.dot_general(L, f, ct10,
                               preferred_element_type=jnp.float32)
        feat_u = lax.dot_general(f, u_ref[...], ct11,
                                 preferred_element_type=jnp.float32)
        feat_v = lax.dot_general(sr_l, v_ref[...], ct11,
                                 preferred_element_type=jnp.float32) + vb_ref[...]
        x = jax.nn.sigmoid(
            feat_u + lax.dot_general(S, feat_v, ct10,
                                     preferred_element_type=jnp.float32))
        e = lax.dot_general(x, e_ref[...], ct11,
                            preferred_element_type=jnp.float32)
        masked = jnp.where(S > 0, e, -3.0e38)
        emax = jnp.max(masked, axis=0, keepdims=True)
        emax = jnp.where(emax > -1.0e30, emax, 0.0)
        emax_n = lax.dot_general(S, emax, ct11,
                                 preferred_element_type=jnp.float32)
        ex = jnp.exp(e - emax_n)
        denom = lax.dot_general(S, ex, ct00,
                                preferred_element_type=jnp.float32)
        rden = 1.0 / jnp.maximum(denom, 1e-12)
        alpha = ex * lax.dot_general(S, rden, ct10,
                                     preferred_element_type=jnp.float32)
        sr_g = lax.dot_general(S, f * alpha, ct00,
                               preferred_element_type=jnp.float32)
        srw = sr_ref[...]
        sr = (lax.dot_general(sr_g, srw[:, :D], ct11,
                              preferred_element_type=jnp.float32)
              + lax.dot_general(sr_l, srw[:, D:], ct11,
                                preferred_element_type=jnp.float32))
        nrm = jnp.sqrt(jnp.sum(sr * sr, axis=1, keepdims=True))
        o_ref[...] = SCALE * sr / jnp.maximum(nrm, 1e-12)

    return pl.pallas_call(
        body,
        out_shape=jax.ShapeDtypeStruct((B, D), jnp.float32),
    )(feat, seg2d, last2d, fc_u_W, fc_v_W, fc_v_b, fc_e_W, fc_sr_W)


def _score(sr_s, emb):
    vb = 2048
    ct11 = (((1,), (1,)), ((), ()))

    def body(sr_ref, emb_ref, o_ref):
        eb = emb_ref[...]
        inv = 1.0 / jnp.maximum(
            jnp.sqrt(jnp.sum(eb * eb, axis=1, keepdims=True)), 1e-12)
        o_ref[...] = lax.dot_general(sr_ref[...], eb * inv, ct11,
                                     preferred_element_type=jnp.float32)

    return pl.pallas_call(
        body,
        grid=((V + vb - 1) // vb,),
        in_specs=[pl.BlockSpec((B, D), lambda k: (0, 0)),
                  pl.BlockSpec((vb, D), lambda k: (k, 0))],
        out_specs=pl.BlockSpec((B, vb), lambda k: (0, k)),
        out_shape=jax.ShapeDtypeStruct((B, V), jnp.float32),
    )(sr_s, emb)


def kernel(iid, edge_index, edge_weight, segment_ids, last_nodes, emb, W1, W2,
           gru_w_ih, gru_w_hh, gru_b_ih, gru_b_hh, fc_u_W, fc_v_W, fc_v_b,
           fc_e_W, fc_sr_W):
    iid = iid.astype(jnp.int32)
    edge_index = edge_index.astype(jnp.int32)
    feat = _feat_gather(iid, emb)
    num, den = _edge_agg(feat, edge_index.reshape(-1), edge_weight)
    feat_new = _gru(num[0, :N], den[0, :N, None], num[1, :N], den[1, :N, None],
                    feat, W1, W2, gru_w_ih, gru_w_hh,
                    gru_b_ih[None, :], gru_b_hh[None, :])
    sr_s = _readout(feat_new, segment_ids.astype(jnp.int32)[:, None],
                    last_nodes.astype(jnp.int32)[:, None],
                    fc_u_W, fc_v_W, fc_v_b[None, :], fc_e_W, fc_sr_W)
    return _score(sr_s, emb)


# speedup vs baseline: 1.1081x; 1.1081x over previous
import functools

import jax
import jax.numpy as jnp
from jax import lax
from jax.experimental import pallas as pl
from jax.experimental.pallas import tpu as pltpu
from jax.experimental.pallas import tpu_sc as plsc

N = 10000
E = 320000
D = 128
B = 256
V = 100000
SCALE = 12.0

NC = 2
NS = 16
NW = NC * NS

N_PAD = 10240
SLAB = N_PAD // NS
GCHUNK = 80
EDGES_PER_TILE = E // NS
NCHUNK = EDGES_PER_TILE // GCHUNK

_MESH = dict(core_axis_name="c", subcore_axis_name="s", num_cores=NC,
             num_subcores=NS)
_SC_PARAMS = pltpu.CompilerParams(needs_layout_passes=False)
_SC_PARAMS_NT = pltpu.CompilerParams(needs_layout_passes=False,
                                     use_tc_tiling_on_sc=False)

_GDN = lax.GatherDimensionNumbers(offset_dims=(), collapsed_slice_dims=(0,),
                                  start_index_map=(0,))


def _bcast_lane(v16, j):
    idx = jnp.full((16, 1), j, jnp.int32)
    return lax.gather(v16, idx, _GDN, (1,),
                      mode=lax.GatherScatterMode.PROMISE_IN_BOUNDS)


def _feat_gather(iid, emb):
    per_w = N_PAD // NW
    nch = per_w // GCHUNK

    @functools.partial(
        pl.kernel,
        out_type=jax.ShapeDtypeStruct((N, D), jnp.float32),
        mesh=plsc.VectorSubcoreMesh(**_MESH),
        compiler_params=_SC_PARAMS,
        scratch_types=[
            pltpu.VMEM((GCHUNK,), jnp.int32),
            pltpu.VMEM((GCHUNK, D), jnp.float32),
            pltpu.SemaphoreType.DMA,
        ],
    )
    def k(iid_hbm, emb_hbm, feat_hbm, idx_v, rows_v, sem):
        wid = lax.axis_index("s") * NC + lax.axis_index("c")
        base_w = jnp.minimum(wid * per_w, N - per_w)

        def body(j, carry):
            base = base_w + j * GCHUNK
            pltpu.sync_copy(iid_hbm.at[pl.ds(base, GCHUNK)], idx_v)
            pltpu.async_copy(emb_hbm.at[idx_v], rows_v, sem).wait()
            pltpu.sync_copy(rows_v, feat_hbm.at[pl.ds(base, GCHUNK)])
            return carry

        lax.fori_loop(0, nch, body, 0)

    return k(iid, emb)


def _edge_agg(feat, eidx_flat, edge_weight):
    L = 16
    NH = N_PAD // 2
    CG = D // L
    K = 640
    SUP = 2 * K
    NCH = E // K
    GSUB = 128

    featg = feat.reshape(N, CG, L).transpose(1, 0, 2).reshape(CG * N, L)
    zacc = jnp.zeros((NH, L), jnp.float32)
    zden = jnp.zeros((N_PAD // D, D), jnp.float32)

    @functools.partial(
        pl.kernel,
        out_type=(jax.ShapeDtypeStruct((NC, NS, NH, L), jnp.float32),
                  jax.ShapeDtypeStruct((NC, N_PAD // D, D), jnp.float32)),
        mesh=plsc.VectorSubcoreMesh(**_MESH),
        compiler_params=_SC_PARAMS_NT,
        scratch_types=[
            pltpu.VMEM((SUP,), jnp.int32),
            pltpu.VMEM((SUP,), jnp.int32),
            pltpu.VMEM((SUP,), jnp.float32),
            pltpu.VMEM((SUP,), jnp.int32),
            pltpu.VMEM((SUP,), jnp.int32),
            pltpu.VMEM((SUP,), jnp.float32),
            pltpu.VMEM((K, L), jnp.float32),
            pltpu.VMEM((K, L), jnp.float32),
            pltpu.VMEM((NH, L), jnp.float32),
            pltpu.VMEM((N_PAD // D, D), jnp.float32),
            pltpu.SemaphoreType.DMA,
            pltpu.SemaphoreType.DMA,
            pltpu.SemaphoreType.DMA,
        ],
    )
    def k(featg_hbm, eidx_hbm, ew_hbm, zacc_hbm, zden_hbm, num_hbm, den_hbm,
          src_a, dst_a, w_a, src_b, dst_b, w_b, rows_a, rows_b,
          acc_v, den_v, sem_a, sem_b, sem_e):
        c = lax.axis_index("c")
        s = lax.axis_index("s")
        g = s // 2
        h = s % 2
        lo = h * NH
        gN = g * N
        iota16 = lax.iota(jnp.int32, 16)

        pltpu.sync_copy(zacc_hbm, acc_v)
        pltpu.sync_copy(zden_hbm, den_v)

        def stream_issue(sup, src_v, dst_v, w_v):
            base = sup * SUP
            pltpu.async_copy(eidx_hbm.at[pl.ds(c * E + base, SUP)], src_v,
                             sem_e)
            pltpu.async_copy(eidx_hbm.at[pl.ds((1 - c) * E + base, SUP)],
                             dst_v, sem_e)
            pltpu.async_copy(ew_hbm.at[pl.ds(base, SUP)], w_v, sem_e)

        def stream_wait(sup, src_v, dst_v, w_v):
            base = sup * SUP
            pltpu.make_async_copy(eidx_hbm.at[pl.ds(c * E + base, SUP)],
                                  src_v, sem_e).wait()
            pltpu.make_async_copy(eidx_hbm.at[pl.ds((1 - c) * E + base, SUP)],
                                  dst_v, sem_e).wait()
            pltpu.make_async_copy(ew_hbm.at[pl.ds(base, SUP)], w_v,
                                  sem_e).wait()

            def off(j, carry):
                sl = pl.ds(j * L, L)
                src_v[sl] = src_v[sl] + gN
                return carry

            lax.fori_loop(0, SUP // L, off, 0)

        def issue(chunk, src_v, rows, sem):
            o = (chunk % 2) * K
            for j in range(K // GSUB):
                pltpu.async_copy(
                    featg_hbm.at[src_v.at[pl.ds(o + j * GSUB, GSUB)]],
                    rows.at[pl.ds(j * GSUB, GSUB)], sem)

        def drain(src_v, rows, sem):
            for j in range(K // GSUB):
                pltpu.make_async_copy(
                    featg_hbm.at[src_v.at[pl.ds(j * GSUB, GSUB)]],
                    rows.at[pl.ds(j * GSUB, GSUB)], sem).wait()

        def process(chunk, dst_v, w_v, rows):
            o = (chunk % 2) * K

            def grp(g2, carry):
                off = o + g2 * L
                d16 = dst_v[pl.ds(off, L)]
                w16 = w_v[pl.ds(off, L)]
                dstl = d16 - lo
                msk = (d16 >= lo) & (dstl < NH)
                erow = iota16 + g2 * L
                for col in range(L):
                    cold = jnp.full((16,), col, jnp.int32)
                    val = plsc.load_gather(rows, [erow, cold])
                    plsc.addupdate_scatter(acc_v, [dstl, cold], val * w16,
                                           mask=msk)

                @pl.when(s == 0)
                def _():
                    plsc.addupdate_scatter(
                        den_v, [d16 >> 7, d16 & (D - 1)], w16)

                return carry

            lax.fori_loop(0, K // L, grp, 0)

        SA = (src_a, dst_a, w_a)
        SB = (src_b, dst_b, w_b)

        def phase(chunk, cur_set, nxt_set, rows_c, sem_c, rows_n, sem_n,
                  wait_next_super, prefetch_set):
            nxt = chunk + 1

            if prefetch_set is not None:
                psup = (chunk + 2) // 2

                @pl.when(psup * SUP < E)
                def _():
                    stream_issue(psup, *prefetch_set)

            @pl.when(nxt < NCH)
            def _():
                if wait_next_super:
                    stream_wait(nxt // 2, *nxt_set)
                issue(nxt, nxt_set[0], rows_n, sem_n)

            drain(cur_set[0], rows_c, sem_c)
            process(chunk, cur_set[1], cur_set[2], rows_c)

        stream_issue(0, *SA)
        stream_wait(0, *SA)
        issue(0, src_a, rows_a, sem_a)

        def quad(i, carry):
            n0 = 4 * i
            phase(n0, SA, SA, rows_a, sem_a, rows_b, sem_b, False, SB)
            phase(n0 + 1, SA, SB, rows_b, sem_b, rows_a, sem_a, True, None)
            phase(n0 + 2, SB, SB, rows_a, sem_a, rows_b, sem_b, False, SA)
            phase(n0 + 3, SB, SA, rows_b, sem_b, rows_a, sem_a, True, None)
            return carry

        lax.fori_loop(0, NCH // 4, quad, 0)

        pltpu.sync_copy(acc_v, num_hbm.at[c, s])

        @pl.when(s == 0)
        def _():
            pltpu.sync_copy(den_v, den_hbm.at[c])

    num5, den2 = k(featg, eidx_flat, edge_weight, zacc, zden)
    num = (num5.reshape(NC, CG, 2, NH, L).transpose(0, 2, 3, 1, 4)
           .reshape(NC, N_PAD, D))
    return num, den2.reshape(NC, N_PAD)


def _gru(num1, den1, num2, den2, feat, W1, W2, w_ih, w_hh, b_ih, b_hh):
    nb = 2000
    cdim = (((1,), (1,)), ((), ()))

    def body(n1_ref, d1_ref, n2_ref, d2_ref, f_ref, W1_ref, W2_ref,
             wih_ref, whh_ref, bih_ref, bhh_ref, o_ref):
        d1 = d1_ref[...]
        d2 = d2_ref[...]
        h1 = jnp.where(d1 > 0, n1_ref[...] / jnp.maximum(d1, 1e-12), 0.0)
        h2 = jnp.where(d2 > 0, n2_ref[...] / jnp.maximum(d2, 1e-12), 0.0)
        neigh1 = lax.dot_general(h1, W1_ref[...], cdim,
                                 preferred_element_type=jnp.float32)
        neigh2 = lax.dot_general(h2, W2_ref[...], cdim,
                                 preferred_element_type=jnp.float32)
        wih = wih_ref[...]
        gi = (lax.dot_general(neigh1, wih[:, :D], cdim,
                              preferred_element_type=jnp.float32)
              + lax.dot_general(neigh2, wih[:, D:], cdim,
                                preferred_element_type=jnp.float32)
              + bih_ref[...])
        f = f_ref[...]
        gh = lax.dot_general(f, whh_ref[...], cdim,
                             preferred_element_type=jnp.float32) + bhh_ref[...]
        r = jax.nn.sigmoid(gi[:, :D] + gh[:, :D])
        z = jax.nn.sigmoid(gi[:, D:2 * D] + gh[:, D:2 * D])
        ng = jnp.tanh(gi[:, 2 * D:] + r * gh[:, 2 * D:])
        o_ref[...] = (1.0 - z) * ng + z * f

    full = lambda shape: pl.BlockSpec(shape, lambda i: (0, 0))
    blk = lambda shape: pl.BlockSpec(shape, lambda i: (i, 0))
    return pl.pallas_call(
        body,
        grid=(N // nb,),
        in_specs=[blk((nb, D)), blk((nb, 1)), blk((nb, D)), blk((nb, 1)),
                  blk((nb, D)), full((D, D)), full((D, D)),
                  full((3 * D, 2 * D)), full((3 * D, D)),
                  full((1, 3 * D)), full((1, 3 * D))],
        out_specs=blk((nb, D)),
        out_shape=jax.ShapeDtypeStruct((N, D), jnp.float32),
    )(num1, den1, num2, den2, feat, W1, W2, w_ih, w_hh, b_ih, b_hh)


def _readout(feat, seg2d, last2d, fc_u_W, fc_v_W, fc_v_b, fc_e_W, fc_sr_W):
    ct11 = (((1,), (1,)), ((), ()))
    ct00 = (((0,), (0,)), ((), ()))
    ct10 = (((1,), (0,)), ((), ()))

    def body(f_ref, seg_ref, last_ref, u_ref, v_ref, vb_ref, e_ref, sr_ref,
             o_ref):
        f = f_ref[...]
        S = (seg_ref[...] == lax.broadcasted_iota(jnp.int32, (N, B), 1)
             ).astype(jnp.float32)
        L = (last_ref[...] == lax.broadcasted_iota(jnp.int32, (B, N), 1)
             ).astype(jnp.float32)
        sr_l = lax.dot_general(L, f, ct10,
                               preferred_element_type=jnp.float32)
        feat_u = lax.dot_general(f, u_ref[...], ct11,
                                 preferred_element_type=jnp.float32)
        feat_v = lax.dot_general(sr_l, v_ref[...], ct11,
                                 preferred_element_type=jnp.float32) + vb_ref[...]
        x = jax.nn.sigmoid(
            feat_u + lax.dot_general(S, feat_v, ct10,
                                     preferred_element_type=jnp.float32))
        e = lax.dot_general(x, e_ref[...], ct11,
                            preferred_element_type=jnp.float32)
        masked = jnp.where(S > 0, e, -3.0e38)
        emax = jnp.max(masked, axis=0, keepdims=True)
        emax = jnp.where(emax > -1.0e30, emax, 0.0)
        emax_n = lax.dot_general(S, emax, ct11,
                                 preferred_element_type=jnp.float32)
        ex = jnp.exp(e - emax_n)
        denom = lax.dot_general(S, ex, ct00,
                                preferred_element_type=jnp.float32)
        rden = 1.0 / jnp.maximum(denom, 1e-12)
        alpha = ex * lax.dot_general(S, rden, ct10,
                                     preferred_element_type=jnp.float32)
        sr_g = lax.dot_general(S, f * alpha, ct00,
                               preferred_element_type=jnp.float32)
        srw = sr_ref[...]
        sr = (lax.dot_general(sr_g, srw[:, :D], ct11,
                              preferred_element_type=jnp.float32)
              + lax.dot_general(sr_l, srw[:, D:], ct11,
                                preferred_element_type=jnp.float32))
        nrm = jnp.sqrt(jnp.sum(sr * sr, axis=1, keepdims=True))
        o_ref[...] = SCALE * sr / jnp.maximum(nrm, 1e-12)

    return pl.pallas_call(
        body,
        out_shape=jax.ShapeDtypeStruct((B, D), jnp.float32),
    )(feat, seg2d, last2d, fc_u_W, fc_v_W, fc_v_b, fc_e_W, fc_sr_W)


def _score(sr_s, emb):
    vb = 2048
    ct11 = (((1,), (1,)), ((), ()))

    def body(sr_ref, emb_ref, o_ref):
        eb = emb_ref[...]
        inv = 1.0 / jnp.maximum(
            jnp.sqrt(jnp.sum(eb * eb, axis=1, keepdims=True)), 1e-12)
        o_ref[...] = lax.dot_general(sr_ref[...], eb * inv, ct11,
                                     preferred_element_type=jnp.float32)

    return pl.pallas_call(
        body,
        grid=((V + vb - 1) // vb,),
        in_specs=[pl.BlockSpec((B, D), lambda k: (0, 0)),
                  pl.BlockSpec((vb, D), lambda k: (k, 0))],
        out_specs=pl.BlockSpec((B, vb), lambda k: (0, k)),
        out_shape=jax.ShapeDtypeStruct((B, V), jnp.float32),
    )(sr_s, emb)


def kernel(iid, edge_index, edge_weight, segment_ids, last_nodes, emb, W1, W2,
           gru_w_ih, gru_w_hh, gru_b_ih, gru_b_hh, fc_u_W, fc_v_W, fc_v_b,
           fc_e_W, fc_sr_W):
    iid = iid.astype(jnp.int32)
    edge_index = edge_index.astype(jnp.int32)
    feat = _feat_gather(iid, emb)
    num, den = _edge_agg(feat, edge_index.reshape(-1), edge_weight)
    feat_new = _gru(num[0, :N], den[0, :N, None], num[1, :N], den[1, :N, None],
                    feat, W1, W2, gru_w_ih, gru_w_hh,
                    gru_b_ih[None, :], gru_b_hh[None, :])
    sr_s = _readout(feat_new, segment_ids.astype(jnp.int32)[:, None],
                    last_nodes.astype(jnp.int32)[:, None],
                    fc_u_W, fc_v_W, fc_v_b[None, :], fc_e_W, fc_sr_W)
    return _score(sr_s, emb)


# speedup vs baseline: 2.0044x; 1.8089x over previous
import functools

import jax
import jax.numpy as jnp
from jax import lax
from jax.experimental import pallas as pl
from jax.experimental.pallas import tpu as pltpu
from jax.experimental.pallas import tpu_sc as plsc

N = 10000
E = 320000
D = 128
B = 256
V = 100000
SCALE = 12.0

NC = 2
NS = 16
NW = NC * NS

N_PAD = 10240
SLAB = N_PAD // NS
GCHUNK = 80
EDGES_PER_TILE = E // NS
NCHUNK = EDGES_PER_TILE // GCHUNK

_MESH = dict(core_axis_name="c", subcore_axis_name="s", num_cores=NC,
             num_subcores=NS)
_SC_PARAMS = pltpu.CompilerParams(needs_layout_passes=False)
_SC_PARAMS_NT = pltpu.CompilerParams(needs_layout_passes=False,
                                     use_tc_tiling_on_sc=False)

_GDN = lax.GatherDimensionNumbers(offset_dims=(), collapsed_slice_dims=(0,),
                                  start_index_map=(0,))


def _bcast_lane(v16, j):
    idx = jnp.full((16, 1), j, jnp.int32)
    return lax.gather(v16, idx, _GDN, (1,),
                      mode=lax.GatherScatterMode.PROMISE_IN_BOUNDS)


def _feat_gather(iid, emb):
    per_w = N_PAD // NW
    nch = per_w // GCHUNK

    @functools.partial(
        pl.kernel,
        out_type=jax.ShapeDtypeStruct((N, D), jnp.float32),
        mesh=plsc.VectorSubcoreMesh(**_MESH),
        compiler_params=_SC_PARAMS,
        scratch_types=[
            pltpu.VMEM((GCHUNK,), jnp.int32),
            pltpu.VMEM((GCHUNK, D), jnp.float32),
            pltpu.SemaphoreType.DMA,
        ],
    )
    def k(iid_hbm, emb_hbm, feat_hbm, idx_v, rows_v, sem):
        wid = lax.axis_index("s") * NC + lax.axis_index("c")
        base_w = jnp.minimum(wid * per_w, N - per_w)

        def body(j, carry):
            base = base_w + j * GCHUNK
            pltpu.sync_copy(iid_hbm.at[pl.ds(base, GCHUNK)], idx_v)
            pltpu.async_copy(emb_hbm.at[idx_v], rows_v, sem).wait()
            pltpu.sync_copy(rows_v, feat_hbm.at[pl.ds(base, GCHUNK)])
            return carry

        lax.fori_loop(0, nch, body, 0)

    return k(iid, emb)


def _edge_agg(feat, eidx_flat, edge_weight):
    L = 16
    NH = N_PAD // 2
    CG = D // L
    K = 640
    SUP = 2 * K
    NCH = E // K
    GSUB = 128

    featg = feat.reshape(N, CG, L).transpose(1, 0, 2).reshape(CG * N, L)
    zacc = jnp.zeros((NH, L), jnp.float32)
    zden = jnp.zeros((N_PAD // D, D), jnp.float32)

    @functools.partial(
        pl.kernel,
        out_type=(jax.ShapeDtypeStruct((NC, NS, NH, L), jnp.float32),
                  jax.ShapeDtypeStruct((NC, N_PAD // D, D), jnp.float32)),
        mesh=plsc.VectorSubcoreMesh(**_MESH),
        compiler_params=_SC_PARAMS_NT,
        scratch_types=[
            pltpu.VMEM((SUP,), jnp.int32),
            pltpu.VMEM((SUP,), jnp.int32),
            pltpu.VMEM((SUP,), jnp.float32),
            pltpu.VMEM((SUP,), jnp.int32),
            pltpu.VMEM((SUP,), jnp.int32),
            pltpu.VMEM((SUP,), jnp.float32),
            pltpu.VMEM((K, L), jnp.float32),
            pltpu.VMEM((K, L), jnp.float32),
            pltpu.VMEM((NH, L), jnp.float32),
            pltpu.VMEM((N_PAD // D, D), jnp.float32),
            pltpu.SemaphoreType.DMA,
            pltpu.SemaphoreType.DMA,
            pltpu.SemaphoreType.DMA,
        ],
    )
    def k(featg_hbm, eidx_hbm, ew_hbm, zacc_hbm, zden_hbm, num_hbm, den_hbm,
          src_a, dst_a, w_a, src_b, dst_b, w_b, rows_a, rows_b,
          acc_v, den_v, sem_a, sem_b, sem_e):
        c = lax.axis_index("c")
        s = lax.axis_index("s")
        g = s // 2
        h = s % 2
        lo = h * NH
        gN = g * N
        iota16 = lax.iota(jnp.int32, 16)

        pltpu.sync_copy(zacc_hbm, acc_v)
        pltpu.sync_copy(zden_hbm, den_v)

        def stream_issue(sup, src_v, dst_v, w_v):
            base = sup * SUP
            pltpu.async_copy(eidx_hbm.at[pl.ds(c * E + base, SUP)], src_v,
                             sem_e)
            pltpu.async_copy(eidx_hbm.at[pl.ds((1 - c) * E + base, SUP)],
                             dst_v, sem_e)
            pltpu.async_copy(ew_hbm.at[pl.ds(base, SUP)], w_v, sem_e)

        def stream_wait(sup, src_v, dst_v, w_v):
            base = sup * SUP
            pltpu.make_async_copy(eidx_hbm.at[pl.ds(c * E + base, SUP)],
                                  src_v, sem_e).wait()
            pltpu.make_async_copy(eidx_hbm.at[pl.ds((1 - c) * E + base, SUP)],
                                  dst_v, sem_e).wait()
            pltpu.make_async_copy(ew_hbm.at[pl.ds(base, SUP)], w_v,
                                  sem_e).wait()

            def off(j, carry):
                sl = pl.ds(j * L, L)
                src_v[sl] = src_v[sl] + gN
                return carry

            lax.fori_loop(0, SUP // L, off, 0)

        def issue(chunk, src_v, rows, sem):
            o = (chunk % 2) * K
            for j in range(K // GSUB):
                pltpu.async_copy(
                    featg_hbm.at[src_v.at[pl.ds(o + j * GSUB, GSUB)]],
                    rows.at[pl.ds(j * GSUB, GSUB)], sem)

        def drain(src_v, rows, sem):
            for j in range(K // GSUB):
                pltpu.make_async_copy(
                    featg_hbm.at[src_v.at[pl.ds(j * GSUB, GSUB)]],
                    rows.at[pl.ds(j * GSUB, GSUB)], sem).wait()

        def process(chunk, dst_v, w_v, rows):
            o = (chunk % 2) * K

            def grp(g2, carry):
                off = o + g2 * L
                d16 = dst_v[pl.ds(off, L)]
                w16 = w_v[pl.ds(off, L)]
                dstl = d16 - lo
                for j in range(L):
                    de = _bcast_lane(dstl, j)
                    wj = _bcast_lane(w16, j)
                    eb = jnp.full((16,), 0, jnp.int32) + (g2 * L + j)
                    mke = (de >= 0) & (de < NH)
                    val = plsc.load_gather(rows, [eb, iota16])
                    plsc.addupdate_scatter(acc_v, [de, iota16], val * wj,
                                           mask=mke)

                @pl.when(s == 0)
                def _():
                    plsc.addupdate_scatter(
                        den_v, [d16 >> 7, d16 & (D - 1)], w16)

                return carry

            lax.fori_loop(0, K // L, grp, 0)

        SA = (src_a, dst_a, w_a)
        SB = (src_b, dst_b, w_b)

        def phase(chunk, cur_set, nxt_set, rows_c, sem_c, rows_n, sem_n,
                  wait_next_super, prefetch_set):
            nxt = chunk + 1

            if prefetch_set is not None:
                psup = (chunk + 2) // 2

                @pl.when(psup * SUP < E)
                def _():
                    stream_issue(psup, *prefetch_set)

            @pl.when(nxt < NCH)
            def _():
                if wait_next_super:
                    stream_wait(nxt // 2, *nxt_set)
                issue(nxt, nxt_set[0], rows_n, sem_n)

            drain(cur_set[0], rows_c, sem_c)
            process(chunk, cur_set[1], cur_set[2], rows_c)

        stream_issue(0, *SA)
        stream_wait(0, *SA)
        issue(0, src_a, rows_a, sem_a)

        def quad(i, carry):
            n0 = 4 * i
            phase(n0, SA, SA, rows_a, sem_a, rows_b, sem_b, False, SB)
            phase(n0 + 1, SA, SB, rows_b, sem_b, rows_a, sem_a, True, None)
            phase(n0 + 2, SB, SB, rows_a, sem_a, rows_b, sem_b, False, SA)
            phase(n0 + 3, SB, SA, rows_b, sem_b, rows_a, sem_a, True, None)
            return carry

        lax.fori_loop(0, NCH // 4, quad, 0)

        pltpu.sync_copy(acc_v, num_hbm.at[c, s])

        @pl.when(s == 0)
        def _():
            pltpu.sync_copy(den_v, den_hbm.at[c])

    num5, den2 = k(featg, eidx_flat, edge_weight, zacc, zden)
    num = (num5.reshape(NC, CG, 2, NH, L).transpose(0, 2, 3, 1, 4)
           .reshape(NC, N_PAD, D))
    return num, den2.reshape(NC, N_PAD)


def _gru(num1, den1, num2, den2, feat, W1, W2, w_ih, w_hh, b_ih, b_hh):
    nb = 2000
    cdim = (((1,), (1,)), ((), ()))

    def body(n1_ref, d1_ref, n2_ref, d2_ref, f_ref, W1_ref, W2_ref,
             wih_ref, whh_ref, bih_ref, bhh_ref, o_ref):
        d1 = d1_ref[...]
        d2 = d2_ref[...]
        h1 = jnp.where(d1 > 0, n1_ref[...] / jnp.maximum(d1, 1e-12), 0.0)
        h2 = jnp.where(d2 > 0, n2_ref[...] / jnp.maximum(d2, 1e-12), 0.0)
        neigh1 = lax.dot_general(h1, W1_ref[...], cdim,
                                 preferred_element_type=jnp.float32)
        neigh2 = lax.dot_general(h2, W2_ref[...], cdim,
                                 preferred_element_type=jnp.float32)
        wih = wih_ref[...]
        gi = (lax.dot_general(neigh1, wih[:, :D], cdim,
                              preferred_element_type=jnp.float32)
              + lax.dot_general(neigh2, wih[:, D:], cdim,
                                preferred_element_type=jnp.float32)
              + bih_ref[...])
        f = f_ref[...]
        gh = lax.dot_general(f, whh_ref[...], cdim,
                             preferred_element_type=jnp.float32) + bhh_ref[...]
        r = jax.nn.sigmoid(gi[:, :D] + gh[:, :D])
        z = jax.nn.sigmoid(gi[:, D:2 * D] + gh[:, D:2 * D])
        ng = jnp.tanh(gi[:, 2 * D:] + r * gh[:, 2 * D:])
        o_ref[...] = (1.0 - z) * ng + z * f

    full = lambda shape: pl.BlockSpec(shape, lambda i: (0, 0))
    blk = lambda shape: pl.BlockSpec(shape, lambda i: (i, 0))
    return pl.pallas_call(
        body,
        grid=(N // nb,),
        in_specs=[blk((nb, D)), blk((nb, 1)), blk((nb, D)), blk((nb, 1)),
                  blk((nb, D)), full((D, D)), full((D, D)),
                  full((3 * D, 2 * D)), full((3 * D, D)),
                  full((1, 3 * D)), full((1, 3 * D))],
        out_specs=blk((nb, D)),
        out_shape=jax.ShapeDtypeStruct((N, D), jnp.float32),
    )(num1, den1, num2, den2, feat, W1, W2, w_ih, w_hh, b_ih, b_hh)


def _readout(feat, seg2d, last2d, fc_u_W, fc_v_W, fc_v_b, fc_e_W, fc_sr_W):
    ct11 = (((1,), (1,)), ((), ()))
    ct00 = (((0,), (0,)), ((), ()))
    ct10 = (((1,), (0,)), ((), ()))

    def body(f_ref, seg_ref, last_ref, u_ref, v_ref, vb_ref, e_ref, sr_ref,
             o_ref):
        f = f_ref[...]
        S = (seg_ref[...] == lax.broadcasted_iota(jnp.int32, (N, B), 1)
             ).astype(jnp.float32)
        L = (last_ref[...] == lax.broadcasted_iota(jnp.int32, (B, N), 1)
             ).astype(jnp.float32)
        sr_l = lax.dot_general(L, f, ct10,
                               preferred_element_type=jnp.float32)
        feat_u = lax.dot_general(f, u_ref[...], ct11,
                                 preferred_element_type=jnp.float32)
        feat_v = lax.dot_general(sr_l, v_ref[...], ct11,
                                 preferred_element_type=jnp.float32) + vb_ref[...]
        x = jax.nn.sigmoid(
            feat_u + lax.dot_general(S, feat_v, ct10,
                                     preferred_element_type=jnp.float32))
        e = lax.dot_general(x, e_ref[...], ct11,
                            preferred_element_type=jnp.float32)
        masked = jnp.where(S > 0, e, -3.0e38)
        emax = jnp.max(masked, axis=0, keepdims=True)
        emax = jnp.where(emax > -1.0e30, emax, 0.0)
        emax_n = lax.dot_general(S, emax, ct11,
                                 preferred_element_type=jnp.float32)
        ex = jnp.exp(e - emax_n)
        denom = lax.dot_general(S, ex, ct00,
                                preferred_element_type=jnp.float32)
        rden = 1.0 / jnp.maximum(denom, 1e-12)
        alpha = ex * lax.dot_general(S, rden, ct10,
                                     preferred_element_type=jnp.float32)
        sr_g = lax.dot_general(S, f * alpha, ct00,
                               preferred_element_type=jnp.float32)
        srw = sr_ref[...]
        sr = (lax.dot_general(sr_g, srw[:, :D], ct11,
                              preferred_element_type=jnp.float32)
              + lax.dot_general(sr_l, srw[:, D:], ct11,
                                preferred_element_type=jnp.float32))
        nrm = jnp.sqrt(jnp.sum(sr * sr, axis=1, keepdims=True))
        o_ref[...] = SCALE * sr / jnp.maximum(nrm, 1e-12)

    return pl.pallas_call(
        body,
        out_shape=jax.ShapeDtypeStruct((B, D), jnp.float32),
    )(feat, seg2d, last2d, fc_u_W, fc_v_W, fc_v_b, fc_e_W, fc_sr_W)


def _score(sr_s, emb):
    vb = 2048
    ct11 = (((1,), (1,)), ((), ()))

    def body(sr_ref, emb_ref, o_ref):
        eb = emb_ref[...]
        inv = 1.0 / jnp.maximum(
            jnp.sqrt(jnp.sum(eb * eb, axis=1, keepdims=True)), 1e-12)
        o_ref[...] = lax.dot_general(sr_ref[...], eb * inv, ct11,
                                     preferred_element_type=jnp.float32)

    return pl.pallas_call(
        body,
        grid=((V + vb - 1) // vb,),
        in_specs=[pl.BlockSpec((B, D), lambda k: (0, 0)),
                  pl.BlockSpec((vb, D), lambda k: (k, 0))],
        out_specs=pl.BlockSpec((B, vb), lambda k: (0, k)),
        out_shape=jax.ShapeDtypeStruct((B, V), jnp.float32),
    )(sr_s, emb)


def kernel(iid, edge_index, edge_weight, segment_ids, last_nodes, emb, W1, W2,
           gru_w_ih, gru_w_hh, gru_b_ih, gru_b_hh, fc_u_W, fc_v_W, fc_v_b,
           fc_e_W, fc_sr_W):
    iid = iid.astype(jnp.int32)
    edge_index = edge_index.astype(jnp.int32)
    feat = _feat_gather(iid, emb)
    num, den = _edge_agg(feat, edge_index.reshape(-1), edge_weight)
    feat_new = _gru(num[0, :N], den[0, :N, None], num[1, :N], den[1, :N, None],
                    feat, W1, W2, gru_w_ih, gru_w_hh,
                    gru_b_ih[None, :], gru_b_hh[None, :])
    sr_s = _readout(feat_new, segment_ids.astype(jnp.int32)[:, None],
                    last_nodes.astype(jnp.int32)[:, None],
                    fc_u_W, fc_v_W, fc_v_b[None, :], fc_e_W, fc_sr_W)
    return _score(sr_s, emb)


# speedup vs baseline: 2.0320x; 1.0138x over previous
import functools

import jax
import jax.numpy as jnp
from jax import lax
from jax.experimental import pallas as pl
from jax.experimental.pallas import tpu as pltpu
from jax.experimental.pallas import tpu_sc as plsc

N = 10000
E = 320000
D = 128
B = 256
V = 100000
SCALE = 12.0

NC = 2
NS = 16
NW = NC * NS

N_PAD = 10240
SLAB = N_PAD // NS
GCHUNK = 80
EDGES_PER_TILE = E // NS
NCHUNK = EDGES_PER_TILE // GCHUNK

_MESH = dict(core_axis_name="c", subcore_axis_name="s", num_cores=NC,
             num_subcores=NS)
_SC_PARAMS = pltpu.CompilerParams(needs_layout_passes=False)
_SC_PARAMS_NT = pltpu.CompilerParams(needs_layout_passes=False,
                                     use_tc_tiling_on_sc=False)

_GDN = lax.GatherDimensionNumbers(offset_dims=(), collapsed_slice_dims=(0,),
                                  start_index_map=(0,))


def _bcast_lane(v16, j):
    idx = jnp.full((16, 1), j, jnp.int32)
    return lax.gather(v16, idx, _GDN, (1,),
                      mode=lax.GatherScatterMode.PROMISE_IN_BOUNDS)


def _feat_gather(iid, emb):
    per_w = N_PAD // NW
    nch = per_w // GCHUNK

    @functools.partial(
        pl.kernel,
        out_type=jax.ShapeDtypeStruct((N, D), jnp.float32),
        mesh=plsc.VectorSubcoreMesh(**_MESH),
        compiler_params=_SC_PARAMS,
        scratch_types=[
            pltpu.VMEM((GCHUNK,), jnp.int32),
            pltpu.VMEM((GCHUNK, D), jnp.float32),
            pltpu.SemaphoreType.DMA,
        ],
    )
    def k(iid_hbm, emb_hbm, feat_hbm, idx_v, rows_v, sem):
        wid = lax.axis_index("s") * NC + lax.axis_index("c")
        base_w = jnp.minimum(wid * per_w, N - per_w)

        def body(j, carry):
            base = base_w + j * GCHUNK
            pltpu.sync_copy(iid_hbm.at[pl.ds(base, GCHUNK)], idx_v)
            pltpu.async_copy(emb_hbm.at[idx_v], rows_v, sem).wait()
            pltpu.sync_copy(rows_v, feat_hbm.at[pl.ds(base, GCHUNK)])
            return carry

        lax.fori_loop(0, nch, body, 0)

    return k(iid, emb)


def _edge_agg(feat, eidx_flat, edge_weight):
    L = 16
    NH = N_PAD // 2
    CG = D // L
    K = 640
    SUP = 2 * K
    NCH = E // K
    GSUB = 128

    featg = feat.reshape(N, CG, L).transpose(1, 0, 2).reshape(CG * N, L)
    zacc = jnp.zeros((NH, L), jnp.float32)
    zden = jnp.zeros((N_PAD // D, D), jnp.float32)

    @functools.partial(
        pl.kernel,
        out_type=(jax.ShapeDtypeStruct((NC, NS, NH, L), jnp.float32),
                  jax.ShapeDtypeStruct((NC, N_PAD // D, D), jnp.float32)),
        mesh=plsc.VectorSubcoreMesh(**_MESH),
        compiler_params=_SC_PARAMS_NT,
        scratch_types=[
            pltpu.VMEM((SUP,), jnp.int32),
            pltpu.VMEM((SUP,), jnp.int32),
            pltpu.VMEM((SUP,), jnp.float32),
            pltpu.VMEM((SUP,), jnp.int32),
            pltpu.VMEM((SUP,), jnp.int32),
            pltpu.VMEM((SUP,), jnp.float32),
            pltpu.VMEM((K, L), jnp.float32),
            pltpu.VMEM((K, L), jnp.float32),
            pltpu.VMEM((NH, L), jnp.float32),
            pltpu.VMEM((N_PAD // D, D), jnp.float32),
            pltpu.SemaphoreType.DMA,
            pltpu.SemaphoreType.DMA,
            pltpu.SemaphoreType.DMA,
        ],
    )
    def k(featg_hbm, eidx_hbm, ew_hbm, zacc_hbm, zden_hbm, num_hbm, den_hbm,
          src_a, dst_a, w_a, src_b, dst_b, w_b, rows_a, rows_b,
          acc_v, den_v, sem_a, sem_b, sem_e):
        c = lax.axis_index("c")
        s = lax.axis_index("s")
        g = s // 2
        h = s % 2
        lo = h * NH
        gN = g * N
        iota16 = lax.iota(jnp.int32, 16)

        diags = [(iota16 + d) & (L - 1) for d in range(L)]

        pltpu.sync_copy(zacc_hbm, acc_v)
        pltpu.sync_copy(zden_hbm, den_v)

        def stream_issue(sup, src_v, dst_v, w_v):
            base = sup * SUP
            pltpu.async_copy(eidx_hbm.at[pl.ds(c * E + base, SUP)], src_v,
                             sem_e)
            pltpu.async_copy(eidx_hbm.at[pl.ds((1 - c) * E + base, SUP)],
                             dst_v, sem_e)
            pltpu.async_copy(ew_hbm.at[pl.ds(base, SUP)], w_v, sem_e)

        def stream_wait(sup, src_v, dst_v, w_v):
            base = sup * SUP
            pltpu.make_async_copy(eidx_hbm.at[pl.ds(c * E + base, SUP)],
                                  src_v, sem_e).wait()
            pltpu.make_async_copy(eidx_hbm.at[pl.ds((1 - c) * E + base, SUP)],
                                  dst_v, sem_e).wait()
            pltpu.make_async_copy(ew_hbm.at[pl.ds(base, SUP)], w_v,
                                  sem_e).wait()

            def off(j, carry):
                sl = pl.ds(j * L, L)
                src_v[sl] = src_v[sl] + gN
                return carry

            lax.fori_loop(0, SUP // L, off, 0)

        def issue(chunk, src_v, rows, sem):
            o = (chunk % 2) * K
            for j in range(K // GSUB):
                pltpu.async_copy(
                    featg_hbm.at[src_v.at[pl.ds(o + j * GSUB, GSUB)]],
                    rows.at[pl.ds(j * GSUB, GSUB)], sem)

        def drain(src_v, rows, sem):
            for j in range(K // GSUB):
                pltpu.make_async_copy(
                    featg_hbm.at[src_v.at[pl.ds(j * GSUB, GSUB)]],
                    rows.at[pl.ds(j * GSUB, GSUB)], sem).wait()

        def process(chunk, dst_v, w_v, rows):
            o = (chunk % 2) * K

            def grp(g2, carry):
                off = o + g2 * L
                d16 = dst_v[pl.ds(off, L)]
                w16 = w_v[pl.ds(off, L)]
                dstl = d16 - lo
                msk = (d16 >= lo) & (dstl < NH)
                erow = iota16 + g2 * L
                for diag in diags:
                    val = plsc.load_gather(rows, [erow, diag])
                    plsc.addupdate_scatter(acc_v, [dstl, diag], val * w16,
                                           mask=msk)

                @pl.when(s == 0)
                def _():
                    plsc.addupdate_scatter(
                        den_v, [d16 >> 7, d16 & (D - 1)], w16)

                return carry

            lax.fori_loop(0, K // L, grp, 0)

        SA = (src_a, dst_a, w_a)
        SB = (src_b, dst_b, w_b)

        def phase(chunk, cur_set, nxt_set, rows_c, sem_c, rows_n, sem_n,
                  wait_next_super, prefetch_set):
            nxt = chunk + 1

            if prefetch_set is not None:
                psup = (chunk + 2) // 2

                @pl.when(psup * SUP < E)
                def _():
                    stream_issue(psup, *prefetch_set)

            @pl.when(nxt < NCH)
            def _():
                if wait_next_super:
                    stream_wait(nxt // 2, *nxt_set)
                issue(nxt, nxt_set[0], rows_n, sem_n)

            drain(cur_set[0], rows_c, sem_c)
            process(chunk, cur_set[1], cur_set[2], rows_c)

        stream_issue(0, *SA)
        stream_wait(0, *SA)
        issue(0, src_a, rows_a, sem_a)

        def quad(i, carry):
            n0 = 4 * i
            phase(n0, SA, SA, rows_a, sem_a, rows_b, sem_b, False, SB)
            phase(n0 + 1, SA, SB, rows_b, sem_b, rows_a, sem_a, True, None)
            phase(n0 + 2, SB, SB, rows_a, sem_a, rows_b, sem_b, False, SA)
            phase(n0 + 3, SB, SA, rows_b, sem_b, rows_a, sem_a, True, None)
            return carry

        lax.fori_loop(0, NCH // 4, quad, 0)

        pltpu.sync_copy(acc_v, num_hbm.at[c, s])

        @pl.when(s == 0)
        def _():
            pltpu.sync_copy(den_v, den_hbm.at[c])

    num5, den2 = k(featg, eidx_flat, edge_weight, zacc, zden)
    num = (num5.reshape(NC, CG, 2, NH, L).transpose(0, 2, 3, 1, 4)
           .reshape(NC, N_PAD, D))
    return num, den2.reshape(NC, N_PAD)


def _gru(num1, den1, num2, den2, feat, W1, W2, w_ih, w_hh, b_ih, b_hh):
    nb = 2000
    cdim = (((1,), (1,)), ((), ()))

    def body(n1_ref, d1_ref, n2_ref, d2_ref, f_ref, W1_ref, W2_ref,
             wih_ref, whh_ref, bih_ref, bhh_ref, o_ref):
        d1 = d1_ref[...]
        d2 = d2_ref[...]
        h1 = jnp.where(d1 > 0, n1_ref[...] / jnp.maximum(d1, 1e-12), 0.0)
        h2 = jnp.where(d2 > 0, n2_ref[...] / jnp.maximum(d2, 1e-12), 0.0)
        neigh1 = lax.dot_general(h1, W1_ref[...], cdim,
                                 preferred_element_type=jnp.float32)
        neigh2 = lax.dot_general(h2, W2_ref[...], cdim,
                                 preferred_element_type=jnp.float32)
        wih = wih_ref[...]
        gi = (lax.dot_general(neigh1, wih[:, :D], cdim,
                              preferred_element_type=jnp.float32)
              + lax.dot_general(neigh2, wih[:, D:], cdim,
                                preferred_element_type=jnp.float32)
              + bih_ref[...])
        f = f_ref[...]
        gh = lax.dot_general(f, whh_ref[...], cdim,
                             preferred_element_type=jnp.float32) + bhh_ref[...]
        r = jax.nn.sigmoid(gi[:, :D] + gh[:, :D])
        z = jax.nn.sigmoid(gi[:, D:2 * D] + gh[:, D:2 * D])
        ng = jnp.tanh(gi[:, 2 * D:] + r * gh[:, 2 * D:])
        o_ref[...] = (1.0 - z) * ng + z * f

    full = lambda shape: pl.BlockSpec(shape, lambda i: (0, 0))
    blk = lambda shape: pl.BlockSpec(shape, lambda i: (i, 0))
    return pl.pallas_call(
        body,
        grid=(N // nb,),
        in_specs=[blk((nb, D)), blk((nb, 1)), blk((nb, D)), blk((nb, 1)),
                  blk((nb, D)), full((D, D)), full((D, D)),
                  full((3 * D, 2 * D)), full((3 * D, D)),
                  full((1, 3 * D)), full((1, 3 * D))],
        out_specs=blk((nb, D)),
        out_shape=jax.ShapeDtypeStruct((N, D), jnp.float32),
    )(num1, den1, num2, den2, feat, W1, W2, w_ih, w_hh, b_ih, b_hh)


def _readout(feat, seg2d, last2d, fc_u_W, fc_v_W, fc_v_b, fc_e_W, fc_sr_W):
    ct11 = (((1,), (1,)), ((), ()))
    ct00 = (((0,), (0,)), ((), ()))
    ct10 = (((1,), (0,)), ((), ()))

    def body(f_ref, seg_ref, last_ref, u_ref, v_ref, vb_ref, e_ref, sr_ref,
             o_ref):
        f = f_ref[...]
        S = (seg_ref[...] == lax.broadcasted_iota(jnp.int32, (N, B), 1)
             ).astype(jnp.float32)
        L = (last_ref[...] == lax.broadcasted_iota(jnp.int32, (B, N), 1)
             ).astype(jnp.float32)
        sr_l = lax.dot_general(L, f, ct10,
                               preferred_element_type=jnp.float32)
        feat_u = lax.dot_general(f, u_ref[...], ct11,
                                 preferred_element_type=jnp.float32)
        feat_v = lax.dot_general(sr_l, v_ref[...], ct11,
                                 preferred_element_type=jnp.float32) + vb_ref[...]
        x = jax.nn.sigmoid(
            feat_u + lax.dot_general(S, feat_v, ct10,
                                     preferred_element_type=jnp.float32))
        e = lax.dot_general(x, e_ref[...], ct11,
                            preferred_element_type=jnp.float32)
        masked = jnp.where(S > 0, e, -3.0e38)
        emax = jnp.max(masked, axis=0, keepdims=True)
        emax = jnp.where(emax > -1.0e30, emax, 0.0)
        emax_n = lax.dot_general(S, emax, ct11,
                                 preferred_element_type=jnp.float32)
        ex = jnp.exp(e - emax_n)
        denom = lax.dot_general(S, ex, ct00,
                                preferred_element_type=jnp.float32)
        rden = 1.0 / jnp.maximum(denom, 1e-12)
        alpha = ex * lax.dot_general(S, rden, ct10,
                                     preferred_element_type=jnp.float32)
        sr_g = lax.dot_general(S, f * alpha, ct00,
                               preferred_element_type=jnp.float32)
        srw = sr_ref[...]
        sr = (lax.dot_general(sr_g, srw[:, :D], ct11,
                              preferred_element_type=jnp.float32)
              + lax.dot_general(sr_l, srw[:, D:], ct11,
                                preferred_element_type=jnp.float32))
        nrm = jnp.sqrt(jnp.sum(sr * sr, axis=1, keepdims=True))
        o_ref[...] = SCALE * sr / jnp.maximum(nrm, 1e-12)

    return pl.pallas_call(
        body,
        out_shape=jax.ShapeDtypeStruct((B, D), jnp.float32),
    )(feat, seg2d, last2d, fc_u_W, fc_v_W, fc_v_b, fc_e_W, fc_sr_W)


def _score(sr_s, emb):
    vb = 2048
    ct11 = (((1,), (1,)), ((), ()))

    def body(sr_ref, emb_ref, o_ref):
        eb = emb_ref[...]
        inv = 1.0 / jnp.maximum(
            jnp.sqrt(jnp.sum(eb * eb, axis=1, keepdims=True)), 1e-12)
        o_ref[...] = lax.dot_general(sr_ref[...], eb * inv, ct11,
                                     preferred_element_type=jnp.float32)

    return pl.pallas_call(
        body,
        grid=((V + vb - 1) // vb,),
        in_specs=[pl.BlockSpec((B, D), lambda k: (0, 0)),
                  pl.BlockSpec((vb, D), lambda k: (k, 0))],
        out_specs=pl.BlockSpec((B, vb), lambda k: (0, k)),
        out_shape=jax.ShapeDtypeStruct((B, V), jnp.float32),
    )(sr_s, emb)


def kernel(iid, edge_index, edge_weight, segment_ids, last_nodes, emb, W1, W2,
           gru_w_ih, gru_w_hh, gru_b_ih, gru_b_hh, fc_u_W, fc_v_W, fc_v_b,
           fc_e_W, fc_sr_W):
    iid = iid.astype(jnp.int32)
    edge_index = edge_index.astype(jnp.int32)
    feat = _feat_gather(iid, emb)
    num, den = _edge_agg(feat, edge_index.reshape(-1), edge_weight)
    feat_new = _gru(num[0, :N], den[0, :N, None], num[1, :N], den[1, :N, None],
                    feat, W1, W2, gru_w_ih, gru_w_hh,
                    gru_b_ih[None, :], gru_b_hh[None, :])
    sr_s = _readout(feat_new, segment_ids.astype(jnp.int32)[:, None],
                    last_nodes.astype(jnp.int32)[:, None],
                    fc_u_W, fc_v_W, fc_v_b[None, :], fc_e_W, fc_sr_W)
    return _score(sr_s, emb)


# speedup vs baseline: 2.0328x; 1.0004x over previous
import functools

import jax
import jax.numpy as jnp
from jax import lax
from jax.experimental import pallas as pl
from jax.experimental.pallas import tpu as pltpu
from jax.experimental.pallas import tpu_sc as plsc

N = 10000
E = 320000
D = 128
B = 256
V = 100000
SCALE = 12.0

NC = 2
NS = 16
NW = NC * NS

N_PAD = 10240
GCHUNK = 80

_MESH = dict(core_axis_name="c", subcore_axis_name="s", num_cores=NC,
             num_subcores=NS)
_SC_PARAMS = pltpu.CompilerParams(needs_layout_passes=False)
_SC_PARAMS_NT = pltpu.CompilerParams(needs_layout_passes=False,
                                     use_tc_tiling_on_sc=False)


def _feat_gather(iid, emb):
    per_w = N_PAD // NW
    nch = per_w // GCHUNK

    @functools.partial(
        pl.kernel,
        out_type=jax.ShapeDtypeStruct((N, D), jnp.float32),
        mesh=plsc.VectorSubcoreMesh(**_MESH),
        compiler_params=_SC_PARAMS,
        scratch_types=[
            pltpu.VMEM((GCHUNK,), jnp.int32),
            pltpu.VMEM((GCHUNK, D), jnp.float32),
            pltpu.SemaphoreType.DMA,
        ],
    )
    def k(iid_hbm, emb_hbm, feat_hbm, idx_v, rows_v, sem):
        wid = lax.axis_index("s") * NC + lax.axis_index("c")
        base_w = jnp.minimum(wid * per_w, N - per_w)

        def body(j, carry):
            base = base_w + j * GCHUNK
            pltpu.sync_copy(iid_hbm.at[pl.ds(base, GCHUNK)], idx_v)
            pltpu.async_copy(emb_hbm.at[idx_v], rows_v, sem).wait()
            pltpu.sync_copy(rows_v, feat_hbm.at[pl.ds(base, GCHUNK)])
            return carry

        lax.fori_loop(0, nch, body, 0)

    return k(iid, emb)


def _edge_agg(feat, eidx_flat, edge_weight):
    L = 16
    NH = N_PAD // 2
    CG = D // L
    K = 640
    SUP = 2 * K
    NCH = E // K
    GSUB = 128

    featg = feat.reshape(N, CG, L).transpose(1, 0, 2).reshape(CG * N, L)
    zacc = jnp.zeros((NH, L), jnp.float32)
    zden = jnp.zeros((N_PAD // D, D), jnp.float32)

    @functools.partial(
        pl.kernel,
        out_type=(jax.ShapeDtypeStruct((NC, NS, NH, L), jnp.float32),
                  jax.ShapeDtypeStruct((NC, N_PAD // D, D), jnp.float32)),
        mesh=plsc.VectorSubcoreMesh(**_MESH),
        compiler_params=_SC_PARAMS_NT,
        scratch_types=[
            pltpu.VMEM((SUP,), jnp.int32),
            pltpu.VMEM((SUP,), jnp.int32),
            pltpu.VMEM((SUP,), jnp.float32),
            pltpu.VMEM((SUP,), jnp.int32),
            pltpu.VMEM((SUP,), jnp.int32),
            pltpu.VMEM((SUP,), jnp.float32),
            pltpu.VMEM((K, L), jnp.float32),
            pltpu.VMEM((K, L), jnp.float32),
            pltpu.VMEM((NH, L), jnp.float32),
            pltpu.VMEM((N_PAD // D, D), jnp.float32),
            pltpu.SemaphoreType.DMA,
            pltpu.SemaphoreType.DMA,
            pltpu.SemaphoreType.DMA,
        ],
    )
    def k(featg_hbm, eidx_hbm, ew_hbm, zacc_hbm, zden_hbm, num_hbm, den_hbm,
          src_a, dst_a, w_a, src_b, dst_b, w_b, rows_a, rows_b,
          acc_v, den_v, sem_a, sem_b, sem_e):
        c = lax.axis_index("c")
        s = lax.axis_index("s")
        g = s // 2
        h = s % 2
        lo = h * NH
        gN = g * N
        iota16 = lax.iota(jnp.int32, 16)

        diags = [(iota16 + d) & (L - 1) for d in range(L)]

        pltpu.sync_copy(zacc_hbm, acc_v)
        pltpu.sync_copy(zden_hbm, den_v)

        def stream_issue(sup, src_v, dst_v, w_v):
            base = sup * SUP
            pltpu.async_copy(eidx_hbm.at[pl.ds(c * E + base, SUP)], src_v,
                             sem_e)
            pltpu.async_copy(eidx_hbm.at[pl.ds((1 - c) * E + base, SUP)],
                             dst_v, sem_e)
            pltpu.async_copy(ew_hbm.at[pl.ds(base, SUP)], w_v, sem_e)

        def stream_wait(sup, src_v, dst_v, w_v):
            base = sup * SUP
            pltpu.make_async_copy(eidx_hbm.at[pl.ds(c * E + base, SUP)],
                                  src_v, sem_e).wait()
            pltpu.make_async_copy(eidx_hbm.at[pl.ds((1 - c) * E + base, SUP)],
                                  dst_v, sem_e).wait()
            pltpu.make_async_copy(ew_hbm.at[pl.ds(base, SUP)], w_v,
                                  sem_e).wait()

            def off(j, carry):
                sl = pl.ds(j * L, L)
                src_v[sl] = src_v[sl] + gN
                return carry

            lax.fori_loop(0, SUP // L, off, 0)

        def issue(chunk, src_v, rows, sem):
            o = (chunk % 2) * K
            for j in range(K // GSUB):
                pltpu.async_copy(
                    featg_hbm.at[src_v.at[pl.ds(o + j * GSUB, GSUB)]],
                    rows.at[pl.ds(j * GSUB, GSUB)], sem)

        def drain(src_v, rows, sem):
            for j in range(K // GSUB):
                pltpu.make_async_copy(
                    featg_hbm.at[src_v.at[pl.ds(j * GSUB, GSUB)]],
                    rows.at[pl.ds(j * GSUB, GSUB)], sem).wait()

        def process(chunk, dst_v, w_v, rows):
            o = (chunk % 2) * K

            def grp(g2, carry):
                off = o + g2 * L
                d16 = dst_v[pl.ds(off, L)]
                w16 = w_v[pl.ds(off, L)]
                dstl = d16 - lo
                msk = (d16 >= lo) & (dstl < NH)
                erow = iota16 + g2 * L
                for diag in diags:
                    val = plsc.load_gather(rows, [erow, diag])
                    plsc.addupdate_scatter(acc_v, [dstl, diag], val * w16,
                                           mask=msk)

                @pl.when(s == 0)
                def _():
                    plsc.addupdate_scatter(
                        den_v, [d16 >> 7, d16 & (D - 1)], w16)

                return carry

            lax.fori_loop(0, K // L, grp, 0)

        SA = (src_a, dst_a, w_a)
        SB = (src_b, dst_b, w_b)

        def phase(chunk, cur_set, nxt_set, rows_c, sem_c, rows_n, sem_n,
                  wait_next_super, prefetch_set):
            nxt = chunk + 1

            if prefetch_set is not None:
                psup = (chunk + 2) // 2

                @pl.when(psup * SUP < E)
                def _():
                    stream_issue(psup, *prefetch_set)

            @pl.when(nxt < NCH)
            def _():
                if wait_next_super:
                    stream_wait(nxt // 2, *nxt_set)
                issue(nxt, nxt_set[0], rows_n, sem_n)

            drain(cur_set[0], rows_c, sem_c)
            process(chunk, cur_set[1], cur_set[2], rows_c)

        stream_issue(0, *SA)
        stream_wait(0, *SA)
        issue(0, src_a, rows_a, sem_a)

        def quad(i, carry):
            n0 = 4 * i
            phase(n0, SA, SA, rows_a, sem_a, rows_b, sem_b, False, SB)
            phase(n0 + 1, SA, SB, rows_b, sem_b, rows_a, sem_a, True, None)
            phase(n0 + 2, SB, SB, rows_a, sem_a, rows_b, sem_b, False, SA)
            phase(n0 + 3, SB, SA, rows_b, sem_b, rows_a, sem_a, True, None)
            return carry

        lax.fori_loop(0, NCH // 4, quad, 0)

        pltpu.sync_copy(acc_v, num_hbm.at[c, s])

        @pl.when(s == 0)
        def _():
            pltpu.sync_copy(den_v, den_hbm.at[c])

    num5, den2 = k(featg, eidx_flat, edge_weight, zacc, zden)
    num = (num5.reshape(NC, CG, 2, NH, L).transpose(0, 2, 3, 1, 4)
           .reshape(NC, N_PAD, D))
    return num, den2.reshape(NC, N_PAD)


def _gru(num1, den1, num2, den2, feat, W1, W2, w_ih, w_hh, b_ih, b_hh):
    nb = 2000
    cdim = (((1,), (1,)), ((), ()))

    def body(n1_ref, d1_ref, n2_ref, d2_ref, f_ref, W1_ref, W2_ref,
             wih_ref, whh_ref, bih_ref, bhh_ref, o_ref):
        d1 = d1_ref[...]
        d2 = d2_ref[...]
        h1 = jnp.where(d1 > 0, n1_ref[...] / jnp.maximum(d1, 1e-12), 0.0)
        h2 = jnp.where(d2 > 0, n2_ref[...] / jnp.maximum(d2, 1e-12), 0.0)
        neigh1 = lax.dot_general(h1, W1_ref[...], cdim,
                                 preferred_element_type=jnp.float32)
        neigh2 = lax.dot_general(h2, W2_ref[...], cdim,
                                 preferred_element_type=jnp.float32)
        wih = wih_ref[...]
        gi = (lax.dot_general(neigh1, wih[:, :D], cdim,
                              preferred_element_type=jnp.float32)
              + lax.dot_general(neigh2, wih[:, D:], cdim,
                                preferred_element_type=jnp.float32)
              + bih_ref[...])
        f = f_ref[...]
        gh = lax.dot_general(f, whh_ref[...], cdim,
                             preferred_element_type=jnp.float32) + bhh_ref[...]
        r = jax.nn.sigmoid(gi[:, :D] + gh[:, :D])
        z = jax.nn.sigmoid(gi[:, D:2 * D] + gh[:, D:2 * D])
        ng = jnp.tanh(gi[:, 2 * D:] + r * gh[:, 2 * D:])
        o_ref[...] = (1.0 - z) * ng + z * f

    full = lambda shape: pl.BlockSpec(shape, lambda i: (0, 0))
    blk = lambda shape: pl.BlockSpec(shape, lambda i: (i, 0))
    return pl.pallas_call(
        body,
        grid=(N // nb,),
        in_specs=[blk((nb, D)), blk((nb, 1)), blk((nb, D)), blk((nb, 1)),
                  blk((nb, D)), full((D, D)), full((D, D)),
                  full((3 * D, 2 * D)), full((3 * D, D)),
                  full((1, 3 * D)), full((1, 3 * D))],
        out_specs=blk((nb, D)),
        out_shape=jax.ShapeDtypeStruct((N, D), jnp.float32),
    )(num1, den1, num2, den2, feat, W1, W2, w_ih, w_hh, b_ih, b_hh)


def _readout(feat, seg2d, last2d, fc_u_W, fc_v_W, fc_v_b, fc_e_W, fc_sr_W):
    ct11 = (((1,), (1,)), ((), ()))
    ct00 = (((0,), (0,)), ((), ()))
    ct10 = (((1,), (0,)), ((), ()))

    def body(f_ref, seg_ref, last_ref, u_ref, v_ref, vb_ref, e_ref, sr_ref,
             o_ref):
        f = f_ref[...]
        S = (seg_ref[...] == lax.broadcasted_iota(jnp.int32, (N, B), 1)
             ).astype(jnp.float32)
        L = (last_ref[...] == lax.broadcasted_iota(jnp.int32, (B, N), 1)
             ).astype(jnp.float32)
        sr_l = lax.dot_general(L, f, ct10,
                               preferred_element_type=jnp.float32)
        feat_u = lax.dot_general(f, u_ref[...], ct11,
                                 preferred_element_type=jnp.float32)
        feat_v = lax.dot_general(sr_l, v_ref[...], ct11,
                                 preferred_element_type=jnp.float32) + vb_ref[...]
        x = jax.nn.sigmoid(
            feat_u + lax.dot_general(S, feat_v, ct10,
                                     preferred_element_type=jnp.float32))
        e = lax.dot_general(x, e_ref[...], ct11,
                            preferred_element_type=jnp.float32)
        masked = jnp.where(S > 0, e, -3.0e38)
        emax = jnp.max(masked, axis=0, keepdims=True)
        emax = jnp.where(emax > -1.0e30, emax, 0.0)
        emax_n = lax.dot_general(S, emax, ct11,
                                 preferred_element_type=jnp.float32)
        ex = jnp.exp(e - emax_n)
        denom = lax.dot_general(S, ex, ct00,
                                preferred_element_type=jnp.float32)
        rden = 1.0 / jnp.maximum(denom, 1e-12)
        alpha = ex * lax.dot_general(S, rden, ct10,
                                     preferred_element_type=jnp.float32)
        sr_g = lax.dot_general(S, f * alpha, ct00,
                               preferred_element_type=jnp.float32)
        srw = sr_ref[...]
        sr = (lax.dot_general(sr_g, srw[:, :D], ct11,
                              preferred_element_type=jnp.float32)
              + lax.dot_general(sr_l, srw[:, D:], ct11,
                                preferred_element_type=jnp.float32))
        nrm = jnp.sqrt(jnp.sum(sr * sr, axis=1, keepdims=True))
        o_ref[...] = SCALE * sr / jnp.maximum(nrm, 1e-12)

    return pl.pallas_call(
        body,
        out_shape=jax.ShapeDtypeStruct((B, D), jnp.float32),
    )(feat, seg2d, last2d, fc_u_W, fc_v_W, fc_v_b, fc_e_W, fc_sr_W)


def _score(sr_s, emb):
    vb = 2048
    ct11 = (((1,), (1,)), ((), ()))

    def body(sr_ref, emb_ref, o_ref):
        eb = emb_ref[...]
        inv = 1.0 / jnp.maximum(
            jnp.sqrt(jnp.sum(eb * eb, axis=1, keepdims=True)), 1e-12)
        o_ref[...] = lax.dot_general(sr_ref[...], eb * inv, ct11,
                                     preferred_element_type=jnp.float32)

    return pl.pallas_call(
        body,
        grid=((V + vb - 1) // vb,),
        in_specs=[pl.BlockSpec((B, D), lambda k: (0, 0)),
                  pl.BlockSpec((vb, D), lambda k: (k, 0))],
        out_specs=pl.BlockSpec((B, vb), lambda k: (0, k)),
        out_shape=jax.ShapeDtypeStruct((B, V), jnp.float32),
    )(sr_s, emb)


def kernel(iid, edge_index, edge_weight, segment_ids, last_nodes, emb, W1, W2,
           gru_w_ih, gru_w_hh, gru_b_ih, gru_b_hh, fc_u_W, fc_v_W, fc_v_b,
           fc_e_W, fc_sr_W):
    iid = iid.astype(jnp.int32)
    edge_index = edge_index.astype(jnp.int32)
    feat = _feat_gather(iid, emb)
    num, den = _edge_agg(feat, edge_index.reshape(-1), edge_weight)
    feat_new = _gru(num[0, :N], den[0, :N, None], num[1, :N], den[1, :N, None],
                    feat, W1, W2, gru_w_ih, gru_w_hh,
                    gru_b_ih[None, :], gru_b_hh[None, :])
    sr_s = _readout(feat_new, segment_ids.astype(jnp.int32)[:, None],
                    last_nodes.astype(jnp.int32)[:, None],
                    fc_u_W, fc_v_W, fc_v_b[None, :], fc_e_W, fc_sr_W)
    return _score(sr_s, emb)
